# baseline (speedup 1.0000x reference)
import jax
import jax.numpy as jnp
from jax import lax
from jax.experimental import pallas as pl
from jax.experimental.pallas import tpu as pltpu

B, N = 16, 4096
X_DIM, Y_DIM, H_DIM, R_DIM = 16, 16, 64, 64
PACK = 256 // X_DIM
GBAT = 2
STEPS = B // GBAT
SROWS = GBAT * N // PACK
BROWS = N // PACK
HW = PACK * H_DIM
NPAIR = PACK // 2


def _bd_mask(rows, cols, rblk, cblk):
    ri = lax.broadcasted_iota(jnp.int32, (rows, cols), 0) // rblk
    ci = lax.broadcasted_iota(jnp.int32, (rows, cols), 1) // cblk
    return (ri == ci).astype(jnp.float32)


def _unpack(v):
    lo = lax.bitcast_convert_type(v << 16, jnp.float32)
    hi = lax.bitcast_convert_type(
        v & jnp.int32(-65536), jnp.float32)
    return lo.astype(jnp.bfloat16), hi.astype(jnp.bfloat16)


def _body(x_ref, y_ref, m_ref, wo_ref, b1_ref, w2_ref, b2_ref, w3_ref,
          b3_ref, out_ref, wall_ref):
    step = pl.program_id(0)

    @pl.when(step == 0)
    def _build():
        wo = wo_ref[...]
        for q in range(4):
            blk = (jnp.tile(wo[8 * q:8 * q + 8], (PACK, PACK))
                   * _bd_mask(128, HW, 8, H_DIM))
            wall_ref[pl.ds(128 * q, 128), :] = blk.astype(jnp.bfloat16)

    b1t = jnp.tile(b1_ref[...], (1, PACK))
    b2t = jnp.tile(b2_ref[...], (1, 2))
    e_mat = _bd_mask(PACK, HW, 1, H_DIM)

    xe, xo = _unpack(x_ref[0])
    ye, yo = _unpack(y_ref[0])
    lhs = jnp.concatenate([xe, xo, ye, yo], axis=1)
    mp = m_ref[0]

    h = jnp.dot(lhs, wall_ref[...], preferred_element_type=jnp.float32)
    h = jnp.maximum(h + b1t, 0.0)
    mexp = jnp.dot(mp, e_mat, preferred_element_type=jnp.float32)
    w2_bd = (jnp.tile(w2_ref[...], (2, 2))
             * _bd_mask(128, 128, 64, 64)).astype(jnp.bfloat16)
    acc_a = jnp.zeros((1, 2 * H_DIM), dtype=jnp.float32)
    acc_b = jnp.zeros((1, 2 * H_DIM), dtype=jnp.float32)
    for p in range(NPAIR):
        g = h[:, 128 * p:128 * (p + 1)].astype(jnp.bfloat16)
        h2 = jnp.dot(g, w2_bd, preferred_element_type=jnp.float32)
        h2 = jnp.maximum(h2 + b2t, 0.0)
        hm = h2 * mexp[:, 128 * p:128 * (p + 1)]
        acc_a = acc_a + jnp.sum(hm[:BROWS], axis=0, keepdims=True)
        acc_b = acc_b + jnp.sum(hm[BROWS:], axis=0, keepdims=True)
    cnt_a = jnp.sum(mp[:BROWS])
    cnt_b = jnp.sum(mp[BROWS:])
    w3 = w3_ref[...]
    b3 = b3_ref[...]
    for bi, (acc, cnt) in enumerate(((acc_a, cnt_a), (acc_b, cnt_b))):
        s = acc[:, :H_DIM] + acc[:, H_DIM:]
        r = jnp.dot(s, w3, preferred_element_type=jnp.float32)
        r = r + cnt * b3
        out_ref[0, pl.ds(bi, 1), :] = r / jnp.maximum(cnt, 1.0)


def kernel(x, y, mask, W1, b1, W2, b2, W3, b3):
    def pack(a):
        ab = a.astype(jnp.bfloat16).reshape(STEPS, SROWS, 128, 2)
        return lax.bitcast_convert_type(ab, jnp.int32)

    xp = pack(x)
    yp = pack(y)
    mp = mask.astype(jnp.float32).reshape(STEPS, SROWS, PACK)
    worder = jnp.concatenate(
        [W1[0:X_DIM:2], W1[1:X_DIM:2],
         W1[X_DIM::2], W1[X_DIM + 1::2]], axis=0)
    b1r = b1.reshape(1, H_DIM)
    b2r = b2.reshape(1, H_DIM)
    b3r = b3.reshape(1, R_DIM)

    out = pl.pallas_call(
        _body,
        grid=(STEPS,),
        in_specs=[
            pl.BlockSpec((1, SROWS, 128), lambda s: (s, 0, 0)),
            pl.BlockSpec((1, SROWS, 128), lambda s: (s, 0, 0)),
            pl.BlockSpec((1, SROWS, PACK), lambda s: (s, 0, 0)),
            pl.BlockSpec((X_DIM + Y_DIM, H_DIM), lambda s: (0, 0)),
            pl.BlockSpec((1, H_DIM), lambda s: (0, 0)),
            pl.BlockSpec((H_DIM, H_DIM), lambda s: (0, 0)),
            pl.BlockSpec((1, H_DIM), lambda s: (0, 0)),
            pl.BlockSpec((H_DIM, R_DIM), lambda s: (0, 0)),
            pl.BlockSpec((1, R_DIM), lambda s: (0, 0)),
        ],
        out_specs=pl.BlockSpec((1, GBAT, R_DIM), lambda s: (s, 0, 0)),
        out_shape=jax.ShapeDtypeStruct((STEPS, GBAT, R_DIM), jnp.float32),
        scratch_shapes=[
            pltpu.VMEM((512, HW), jnp.bfloat16),
        ],
        compiler_params=pltpu.CompilerParams(
            dimension_semantics=("arbitrary",),
        ),
    )(xp, yp, mp, worder, b1r, W2, b2r, W3, b3r)
    return out.reshape(B, R_DIM)


# speedup vs baseline: 6.1075x; 6.1075x over previous
import jax
import jax.numpy as jnp
from jax import lax
from jax.experimental import pallas as pl
from jax.experimental.pallas import tpu as pltpu

B, N = 16, 4096
X_DIM, Y_DIM, H_DIM, R_DIM = 16, 16, 64, 64
PACK = 128 // X_DIM
PROWS = N // PACK
NPAIR = PACK // 2


def _bd_mask(rows, cols, rblk, cblk):
    ri = lax.broadcasted_iota(jnp.int32, (rows, cols), 0) // rblk
    ci = lax.broadcasted_iota(jnp.int32, (rows, cols), 1) // cblk
    return (ri == ci).astype(jnp.float32)


def _body(x_ref, y_ref, m_ref, w1_ref, b1_ref, w2_ref, b2_ref, w3_ref,
          b3_ref, out_ref):
    w1 = w1_ref[...]
    w1x_bd = (jnp.tile(w1[:X_DIM], (PACK, PACK))
              * _bd_mask(128, 512, 16, 64)).astype(jnp.bfloat16)
    w1y_bd = (jnp.tile(w1[X_DIM:], (PACK, PACK))
              * _bd_mask(128, 512, 16, 64)).astype(jnp.bfloat16)
    w2_bd = (jnp.tile(w2_ref[...], (2, 2))
             * _bd_mask(128, 128, 64, 64)).astype(jnp.bfloat16)
    b1t = jnp.tile(b1_ref[...], (1, PACK))
    b2t = jnp.tile(b2_ref[...], (1, 2))
    e_mat = _bd_mask(PACK, PACK * H_DIM, 1, H_DIM).astype(jnp.bfloat16)

    px = x_ref[0]
    py = y_ref[0]
    mp = m_ref[0]

    h = jnp.dot(px, w1x_bd, preferred_element_type=jnp.float32)
    h = h + jnp.dot(py, w1y_bd, preferred_element_type=jnp.float32)
    h = jnp.maximum(h + b1t, 0.0)
    mexp = jnp.dot(mp, e_mat, preferred_element_type=jnp.float32)
    acc = jnp.zeros((1, 2 * H_DIM), dtype=jnp.float32)
    for p in range(NPAIR):
        g = h[:, 2 * H_DIM * p:2 * H_DIM * (p + 1)].astype(jnp.bfloat16)
        h2 = jnp.dot(g, w2_bd, preferred_element_type=jnp.float32)
        h2 = jnp.maximum(h2 + b2t, 0.0)
        mm = mexp[:, 2 * H_DIM * p:2 * H_DIM * (p + 1)]
        acc = acc + jnp.sum(h2 * mm, axis=0, keepdims=True)
    s = acc[:, :H_DIM] + acc[:, H_DIM:]
    cnt = jnp.sum(mp.astype(jnp.float32))
    r = jnp.dot(s, w3_ref[...], preferred_element_type=jnp.float32)
    r = r + cnt * b3_ref[...]
    out_ref[0] = r / jnp.maximum(cnt, 1.0)


def kernel(x, y, mask, W1, b1, W2, b2, W3, b3):
    xd = x.astype(jnp.bfloat16).reshape(B, PROWS, 128)
    yd = y.astype(jnp.bfloat16).reshape(B, PROWS, 128)
    mp = mask.astype(jnp.bfloat16).reshape(B, PROWS, PACK)
    b1r = b1.reshape(1, H_DIM)
    b2r = b2.reshape(1, H_DIM)
    b3r = b3.reshape(1, R_DIM)

    out = pl.pallas_call(
        _body,
        grid=(B,),
        in_specs=[
            pl.BlockSpec((1, PROWS, 128), lambda b: (b, 0, 0)),
            pl.BlockSpec((1, PROWS, 128), lambda b: (b, 0, 0)),
            pl.BlockSpec((1, PROWS, PACK), lambda b: (b, 0, 0)),
            pl.BlockSpec((X_DIM + Y_DIM, H_DIM), lambda b: (0, 0)),
            pl.BlockSpec((1, H_DIM), lambda b: (0, 0)),
            pl.BlockSpec((H_DIM, H_DIM), lambda b: (0, 0)),
            pl.BlockSpec((1, H_DIM), lambda b: (0, 0)),
            pl.BlockSpec((H_DIM, R_DIM), lambda b: (0, 0)),
            pl.BlockSpec((1, R_DIM), lambda b: (0, 0)),
        ],
        out_specs=pl.BlockSpec((1, 1, R_DIM), lambda b: (b, 0, 0)),
        out_shape=jax.ShapeDtypeStruct((B, 1, R_DIM), jnp.float32),
        compiler_params=pltpu.CompilerParams(
            dimension_semantics=("arbitrary",),
        ),
    )(xd, yd, mp, W1, b1r, W2, b2r, W3, b3r)
    return out.reshape(B, R_DIM)
